# Initial kernel scaffold; baseline (speedup 1.0000x reference)
import jax
import jax.numpy as jnp
from jax.experimental import pallas as pl


def kernel(input, weight):
    raise NotImplementedError("write your pallas kernel here")


# speedup vs baseline: 2.0782x; 2.0782x over previous
import jax
import jax.numpy as jnp
from jax import lax
from jax.experimental import pallas as pl
from jax.experimental.pallas import tpu as pltpu
from jax.experimental.pallas import tpu_sc as plsc

VOCAB_ROWS = 1_000_000
EMB_DIM = 32
NUM_TERMS = 50
NUM_BAGS = 16384
NUM_CORES = 2
NUM_SUBCORES = 16
NUM_WORKERS = NUM_CORES * NUM_SUBCORES
BAGS_PER_WORKER = NUM_BAGS // NUM_WORKERS
CHUNK = 128
CHUNKS_PER_TERM = BAGS_PER_WORKER // CHUNK


def _accumulate(acc, buf, c, first):
    for r in range(CHUNK):
        b = c * CHUNK + r
        for h in (0, 16):
            x = buf[r, pl.ds(h, 16)]
            if first:
                acc[b, pl.ds(h, 16)] = x
            else:
                plsc.addupdate(acc.at[b, pl.ds(h, 16)], x)


def _bag_sum_body(idx_hbm, weight_hbm, out_hbm, idx_v, buf0, buf1, acc,
                  sem0, sem1):
    wid = lax.axis_index("s") * NUM_CORES + lax.axis_index("c")
    base = wid * BAGS_PER_WORKER

    pltpu.sync_copy(idx_hbm.at[:, pl.ds(wid * CHUNKS_PER_TERM,
                                        CHUNKS_PER_TERM)], idx_v)

    bufs = (buf0, buf1)
    sems = (sem0, sem1)

    def start(l, c, parity):
        pltpu.async_copy(weight_hbm.at[idx_v.at[l, c]], bufs[parity],
                         sems[parity])

    def wait(parity):
        pltpu.make_async_copy(weight_hbm.at[pl.ds(0, CHUNK)], bufs[parity],
                              sems[parity]).wait()

    start(0, 0, 0)

    @pl.loop(0, NUM_TERMS)
    def _(l):
        first = l == 0
        for c in range(CHUNKS_PER_TERM):
            parity = c % 2
            if c + 1 < CHUNKS_PER_TERM:
                start(l, c + 1, 1 - parity)
            else:
                @pl.when(l + 1 < NUM_TERMS)
                def _():
                    start(l + 1, 0, 1 - parity)
            wait(parity)

            @pl.when(first)
            def _():
                _accumulate(acc, bufs[parity], c, True)

            @pl.when(jnp.logical_not(first))
            def _():
                _accumulate(acc, bufs[parity], c, False)

    pltpu.sync_copy(acc, out_hbm.at[pl.ds(base, BAGS_PER_WORKER)])


def kernel(input, weight):
    idx = input.astype(jnp.int32).reshape(NUM_TERMS, NUM_BAGS // CHUNK, CHUNK)
    mesh = plsc.VectorSubcoreMesh(core_axis_name="c", subcore_axis_name="s")
    run = pl.kernel(
        _bag_sum_body,
        out_type=jax.ShapeDtypeStruct((NUM_BAGS, EMB_DIM), jnp.float32),
        mesh=mesh,
        compiler_params=pltpu.CompilerParams(use_tc_tiling_on_sc=False),
        scratch_types=[
            pltpu.VMEM((NUM_TERMS, CHUNKS_PER_TERM, CHUNK), jnp.int32),
            pltpu.VMEM((CHUNK, EMB_DIM), jnp.float32),
            pltpu.VMEM((CHUNK, EMB_DIM), jnp.float32),
            pltpu.VMEM((BAGS_PER_WORKER, EMB_DIM), jnp.float32),
            pltpu.SemaphoreType.DMA,
            pltpu.SemaphoreType.DMA,
        ],
    )
    return run(idx, weight)
